# baseline (speedup 1.0000x reference)
import jax
import jax.numpy as jnp
from jax import lax
from jax.experimental import pallas as pl
from jax.experimental.pallas import tpu as pltpu
from jax.experimental.pallas import tpu_sc as plsc

_N = 5000
_L = 16
_NW = 32
_NSC = 16
_RPW = 160
_NP = _NW * _RPW
_NC = _NP // _L
_NB = _RPW // _L
_BPS = _NP // _NSC
_KX = 32
_KY = 16
_K2 = _KX * _KY
_SX = _KX / 800.0
_SY = _KY / 800.0
_RU = 4
_THR = 0.4


def _decay_body(x0h, y0h, x1h, y1h, sch, outh,
                x0v, y0v, x1v, y1v, scv,
                gx0v, gy0v, gx1v, gy1v, gscv, arv, kbv,
                bidv, posv, origv, sidxv, oidxv, outv,
                histv, offv, allhv,
                pb0, pb1, pb2, pb3, ob0, ob1,
                hist_s, offs_s, gofs_s,
                sh_hist, sh_sidx, sem):
    cid = lax.axis_index("c")
    sid = lax.axis_index("s")
    wid = sid * 2 + cid

    pltpu.sync_copy(x0h, x0v)
    pltpu.sync_copy(y0h, y0v)
    pltpu.sync_copy(x1h, x1v)
    pltpu.sync_copy(y1h, y1v)
    pltpu.sync_copy(sch, scv)

    lanes = lax.iota(jnp.int32, _L)
    ones = jnp.full((_L,), 1.0, jnp.float32)
    zi = jnp.zeros((_L,), jnp.int32)

    def _bfly(v, op):
        for s in (1, 2, 4, 8):
            v = op(v, v.at[lanes ^ s].get(mode="promise_in_bounds"))
        return v

    def _bucketx(v):
        return jnp.clip((v * _SX).astype(jnp.int32), 0, _KX - 1)

    def _buckety(v):
        return jnp.clip((v * _SY).astype(jnp.int32), 0, _KY - 1)

    sbase = sid * _BPS

    def bid_chunk(k, carry):
        sl = pl.ds(k * _L, _L)
        gsl = pl.ds(sbase + k * _L, _L)
        bidv[sl] = _bucketx(x0v[gsl]) * _KY + _buckety(y0v[gsl])
        return carry
    lax.fori_loop(0, _BPS // _L, bid_chunk, 0)

    def hist_zero(c, carry):
        for l in range(_L):
            hist_s[c * _L + l] = jnp.int32(0)
        return carry
    lax.fori_loop(0, _K2 // _L, hist_zero, 0)

    def hist_acc(k, carry):
        v = bidv[pl.ds(k * _L, _L)]
        for l in range(_L):
            b = v[l]
            hist_s[b] = hist_s[b] + 1
        return carry
    lax.fori_loop(0, _BPS // _L, hist_acc, 0)

    def hist_pub(c, carry):
        acc = zi
        for l in range(_L):
            acc = jnp.where(lanes == l, hist_s[c * _L + l], acc)
        histv[pl.ds(c * _L, _L)] = acc
        return carry
    lax.fori_loop(0, _K2 // _L, hist_pub, 0)
    pltpu.sync_copy(histv, sh_hist.at[pl.ds(sid * _K2, _K2)])
    plsc.subcore_barrier()
    pltpu.sync_copy(sh_hist, allhv)

    def off_chunk(c, carry):
        tot = zi
        part = zi
        for w in range(_NSC):
            h = allhv[pl.ds(w * _K2 + c * _L, _L)]
            tot = tot + h
            part = part + jnp.where(w < sid, h, zi)
        incl = tot
        for s in (1, 2, 4, 8):
            sh = incl.at[jnp.maximum(lanes - s, 0)].get(mode="promise_in_bounds")
            incl = incl + jnp.where(lanes >= s, sh, zi)
        excl = incl - tot
        gof = carry + excl
        off = gof + part
        offv[pl.ds(c * _L, _L)] = off
        for l in range(_L):
            offs_s[c * _L + l] = off[l]
            gofs_s[c * _L + l] = gof[l]
        return carry + incl[_L - 1]
    lax.fori_loop(0, _K2 // _L, off_chunk, jnp.int32(0))
    gofs_s[_K2] = jnp.int32(_NP)

    def pos_chunk(k, carry):
        v = bidv[pl.ds(k * _L, _L)]
        pos = zi
        for l in range(_L):
            b = v[l]
            p = offs_s[b]
            offs_s[b] = p + 1
            pos = jnp.where(lanes == l, p, pos)
        posv[pl.ds(k * _L, _L)] = pos
        origv[pl.ds(k * _L, _L)] = sbase + k * _L + lanes
        return carry
    lax.fori_loop(0, _BPS // _L, pos_chunk, 0)

    pbufs = (pb0, pb1, pb2, pb3)
    for q in range(_BPS // _L):
        pbufs[q // 5][pl.ds((q % 5) * _L, _L)] = posv[pl.ds(q * _L, _L)]
    for h in range(4):
        pltpu.async_copy(origv.at[pl.ds(h * 80, 80)], sh_sidx.at[pbufs[h]], sem)
    pltpu.make_async_copy(origv.at[pl.ds(0, _BPS)],
                          sh_sidx.at[pl.ds(0, _BPS)], sem).wait()
    plsc.subcore_barrier()
    pltpu.sync_copy(sh_sidx, sidxv)

    def perm_chunk(k, exts):
        extmx, extmy = exts
        sl = pl.ds(k * _L, _L)
        idx = sidxv[sl]
        gx0 = plsc.load_gather(x0v, [idx])
        gy0 = plsc.load_gather(y0v, [idx])
        gx1 = plsc.load_gather(x1v, [idx])
        gy1 = plsc.load_gather(y1v, [idx])
        gsc = plsc.load_gather(scv, [idx])
        gx0v[sl] = gx0
        gy0v[sl] = gy0
        gx1v[sl] = gx1
        gy1v[sl] = gy1
        gscv[sl] = gsc
        kbv[sl] = _bucketx(gx0) * _KY + _buckety(gy0)
        extx = gx1 - gx0
        exty = gy1 - gy0
        arv[sl] = extx * exty
        return (jnp.maximum(extmx, extx), jnp.maximum(extmy, exty))
    zf = jnp.zeros((_L,), jnp.float32)
    extmx, extmy = lax.fori_loop(0, _NC, perm_chunk, (zf, zf))
    extmx = _bfly(extmx, jnp.maximum)
    extmy = _bfly(extmy, jnp.maximum)

    def row_block(rb, carry):
        blk = wid + _NW * rb
        i0 = blk * _L
        sl_i = pl.ds(i0, _L)
        rx0 = gx0v[sl_i]
        ry0 = gy0v[sl_i]
        rx1 = gx1v[sl_i]
        ry1 = gy1v[sl_i]
        rar = arv[sl_i]
        rsc = gscv[sl_i]
        oidxv[pl.ds(rb * _L, _L)] = sidxv[sl_i]

        xlo = _bucketx(_bfly(rx0, jnp.minimum) - 0.6 * extmx - 0.5)[0]
        xhi = _bucketx(_bfly(0.6 * rx1 + 0.4 * rx0, jnp.maximum) + 0.5)[0]
        ylo = _buckety(_bfly(ry0, jnp.minimum) - 0.6 * extmy - 0.5)[0]
        yhi = _buckety(_bfly(0.6 * ry1 + 0.4 * ry0, jnp.maximum) + 0.5)[0]

        res = jnp.zeros((_L,), jnp.float32)
        for g in range(_L // _RU):
            ix0 = [rx0[g * _RU + r] for r in range(_RU)]
            iy0 = [ry0[g * _RU + r] for r in range(_RU)]
            ix1 = [rx1[g * _RU + r] for r in range(_RU)]
            iy1 = [ry1[g * _RU + r] for r in range(_RU)]
            iar = [rar[g * _RU + r] for r in range(_RU)]
            isc = [rsc[g * _RU + r] for r in range(_RU)]

            def strip(xb, accs):
                cl = xb * _KY + ylo
                ch = xb * _KY + yhi
                e_lo = gofs_s[cl]
                e_hi = gofs_s[ch + 1]
                c_lo = lax.shift_right_logical(e_lo, 4)
                c_hi = lax.shift_right_logical(e_hi + 15, 4)

                def jchunk(k, accs):
                    sl = pl.ds(k * _L, _L)
                    jx0 = gx0v[sl]
                    jy0 = gy0v[sl]
                    jx1 = gx1v[sl]
                    jy1 = gy1v[sl]
                    js = gscv[sl]
                    ja = arv[sl]
                    jkb = kbv[sl]
                    m = (jkb >= cl) & (jkb <= ch)
                    nxt = []
                    for r in range(_RU):
                        wx = jnp.maximum(jnp.minimum(ix1[r], jx1) - jnp.maximum(ix0[r], jx0), 0.0)
                        wy = jnp.maximum(jnp.minimum(iy1[r], jy1) - jnp.maximum(iy0[r], jy0), 0.0)
                        inter = wx * wy
                        union = iar[r] + ja - inter
                        hit = (inter > _THR * union) & (js > isc[r]) & m
                        ratio = (union - inter) / union
                        nxt.append(accs[r] * jnp.where(hit, ratio, 1.0))
                    return tuple(nxt)

                return lax.fori_loop(c_lo, c_hi, jchunk, accs)

            accs = lax.fori_loop(xlo, xhi + 1, strip, (ones,) * _RU)
            for r in range(_RU):
                a = _bfly(accs[r], jnp.multiply)
                res = jnp.where(lanes == (g * _RU + r), a[0], res)
        outv[pl.ds(rb * _L, _L)] = res * rsc
        return carry

    lax.fori_loop(0, _NB, row_block, 0)

    obufs = (ob0, ob1)
    for q in range(_RPW // _L):
        obufs[q // 5][pl.ds((q % 5) * _L, _L)] = oidxv[pl.ds(q * _L, _L)]
    for h in range(2):
        pltpu.async_copy(outv.at[pl.ds(h * 80, 80)], outh.at[obufs[h]], sem)
    pltpu.make_async_copy(outv.at[pl.ds(0, _RPW)],
                          outh.at[pl.ds(0, _RPW)], sem).wait()


_mesh = plsc.VectorSubcoreMesh(core_axis_name="c", subcore_axis_name="s")

_decay_call = pl.kernel(
    _decay_body,
    out_type=jax.ShapeDtypeStruct((_NP,), jnp.float32),
    mesh=_mesh,
    scratch_types=[
        pltpu.VMEM((_NP,), jnp.float32),
        pltpu.VMEM((_NP,), jnp.float32),
        pltpu.VMEM((_NP,), jnp.float32),
        pltpu.VMEM((_NP,), jnp.float32),
        pltpu.VMEM((_NP,), jnp.float32),
        pltpu.VMEM((_NP,), jnp.float32),
        pltpu.VMEM((_NP,), jnp.float32),
        pltpu.VMEM((_NP,), jnp.float32),
        pltpu.VMEM((_NP,), jnp.float32),
        pltpu.VMEM((_NP,), jnp.float32),
        pltpu.VMEM((_NP,), jnp.float32),
        pltpu.VMEM((_NP,), jnp.int32),
        pltpu.VMEM((_BPS,), jnp.int32),
        pltpu.VMEM((_BPS,), jnp.int32),
        pltpu.VMEM((_BPS,), jnp.int32),
        pltpu.VMEM((_NP,), jnp.int32),
        pltpu.VMEM((_RPW,), jnp.int32),
        pltpu.VMEM((_RPW,), jnp.float32),
        pltpu.VMEM((_K2,), jnp.int32),
        pltpu.VMEM((_K2,), jnp.int32),
        pltpu.VMEM((_NSC * _K2,), jnp.int32),
        pltpu.VMEM((80,), jnp.int32),
        pltpu.VMEM((80,), jnp.int32),
        pltpu.VMEM((80,), jnp.int32),
        pltpu.VMEM((80,), jnp.int32),
        pltpu.VMEM((80,), jnp.int32),
        pltpu.VMEM((80,), jnp.int32),
        pltpu.SMEM((_K2,), jnp.int32),
        pltpu.SMEM((_K2,), jnp.int32),
        pltpu.SMEM((_K2 + 1,), jnp.int32),
        pltpu.VMEM_SHARED((_NSC * _K2,), jnp.int32),
        pltpu.VMEM_SHARED((_NP,), jnp.int32),
        pltpu.SemaphoreType.DMA,
    ],
    compiler_params=pltpu.CompilerParams(needs_layout_passes=False),
)


def kernel(boxes, scores):
    pad = _NP - _N
    big = jnp.float32(4.0e8)
    x0 = jnp.concatenate([boxes[:, 0], jnp.full((pad,), big, jnp.float32)])
    y0 = jnp.concatenate([boxes[:, 1], jnp.full((pad,), big, jnp.float32)])
    x1 = jnp.concatenate([boxes[:, 2] + 1.0, jnp.full((pad,), big + 1.0, jnp.float32)])
    y1 = jnp.concatenate([boxes[:, 3] + 1.0, jnp.full((pad,), big + 1.0, jnp.float32)])
    sc = jnp.concatenate([scores, jnp.zeros((pad,), jnp.float32)])
    decayed = _decay_call(x0, y0, x1, y1, sc)
    return decayed[:_N]


# speedup vs baseline: 1.0830x; 1.0830x over previous
import jax
import jax.numpy as jnp
from jax import lax
from jax.experimental import pallas as pl
from jax.experimental.pallas import tpu as pltpu
from jax.experimental.pallas import tpu_sc as plsc

_N = 5000
_L = 16
_NW = 32
_NSC = 16
_RPW = 160
_NP = _NW * _RPW
_NC = _NP // _L
_NB = _RPW // _L
_BPS = _NP // _NSC
_KX = 8
_KY = 64
_K2 = _KX * _KY
_SX = _KX / 800.0
_SY = _KY / 800.0
_RU = 4
_THR = 0.4


def _decay_body(x0h, y0h, x1h, y1h, sch, outh,
                x0v, y0v, x1v, y1v, scv,
                gx0v, gy0v, gx1v, gy1v, gscv, arv, kbv,
                bidv, posv, origv, sidxv, oidxv, outv,
                histv, offv, allhv,
                pb0, pb1, pb2, pb3, ob0, ob1,
                hist_s, offs_s, gofs_s,
                sh_hist, sh_sidx, sem):
    cid = lax.axis_index("c")
    sid = lax.axis_index("s")
    wid = sid * 2 + cid

    pltpu.sync_copy(x0h, x0v)
    pltpu.sync_copy(y0h, y0v)
    pltpu.sync_copy(x1h, x1v)
    pltpu.sync_copy(y1h, y1v)
    pltpu.sync_copy(sch, scv)

    lanes = lax.iota(jnp.int32, _L)
    ones = jnp.full((_L,), 1.0, jnp.float32)
    zi = jnp.zeros((_L,), jnp.int32)

    def _bfly(v, op):
        for s in (1, 2, 4, 8):
            v = op(v, v.at[lanes ^ s].get(mode="promise_in_bounds"))
        return v

    def _bucketx(v):
        return jnp.clip((v * _SX).astype(jnp.int32), 0, _KX - 1)

    def _buckety(v):
        return jnp.clip((v * _SY).astype(jnp.int32), 0, _KY - 1)

    sbase = sid * _BPS

    def bid_chunk(k, carry):
        sl = pl.ds(k * _L, _L)
        gsl = pl.ds(sbase + k * _L, _L)
        bidv[sl] = _bucketx(x0v[gsl]) * _KY + _buckety(y0v[gsl])
        return carry
    lax.fori_loop(0, _BPS // _L, bid_chunk, 0)

    def hist_zero(c, carry):
        for l in range(_L):
            hist_s[c * _L + l] = jnp.int32(0)
        return carry
    lax.fori_loop(0, _K2 // _L, hist_zero, 0)

    def hist_acc(k, carry):
        v = bidv[pl.ds(k * _L, _L)]
        for l in range(_L):
            b = v[l]
            hist_s[b] = hist_s[b] + 1
        return carry
    lax.fori_loop(0, _BPS // _L, hist_acc, 0)

    def hist_pub(c, carry):
        acc = zi
        for l in range(_L):
            acc = jnp.where(lanes == l, hist_s[c * _L + l], acc)
        histv[pl.ds(c * _L, _L)] = acc
        return carry
    lax.fori_loop(0, _K2 // _L, hist_pub, 0)
    pltpu.sync_copy(histv, sh_hist.at[pl.ds(sid * _K2, _K2)])
    plsc.subcore_barrier()
    pltpu.sync_copy(sh_hist, allhv)

    def off_chunk(c, carry):
        tot = zi
        part = zi
        for w in range(_NSC):
            h = allhv[pl.ds(w * _K2 + c * _L, _L)]
            tot = tot + h
            part = part + jnp.where(w < sid, h, zi)
        incl = tot
        for s in (1, 2, 4, 8):
            sh = incl.at[jnp.maximum(lanes - s, 0)].get(mode="promise_in_bounds")
            incl = incl + jnp.where(lanes >= s, sh, zi)
        excl = incl - tot
        gof = carry + excl
        off = gof + part
        offv[pl.ds(c * _L, _L)] = off
        for l in range(_L):
            offs_s[c * _L + l] = off[l]
            gofs_s[c * _L + l] = gof[l]
        return carry + incl[_L - 1]
    lax.fori_loop(0, _K2 // _L, off_chunk, jnp.int32(0))
    gofs_s[_K2] = jnp.int32(_NP)

    def pos_chunk(k, carry):
        v = bidv[pl.ds(k * _L, _L)]
        pos = zi
        for l in range(_L):
            b = v[l]
            p = offs_s[b]
            offs_s[b] = p + 1
            pos = jnp.where(lanes == l, p, pos)
        posv[pl.ds(k * _L, _L)] = pos
        origv[pl.ds(k * _L, _L)] = sbase + k * _L + lanes
        return carry
    lax.fori_loop(0, _BPS // _L, pos_chunk, 0)

    pbufs = (pb0, pb1, pb2, pb3)
    for q in range(_BPS // _L):
        pbufs[q // 5][pl.ds((q % 5) * _L, _L)] = posv[pl.ds(q * _L, _L)]
    for h in range(4):
        pltpu.async_copy(origv.at[pl.ds(h * 80, 80)], sh_sidx.at[pbufs[h]], sem)
    pltpu.make_async_copy(origv.at[pl.ds(0, _BPS)],
                          sh_sidx.at[pl.ds(0, _BPS)], sem).wait()
    plsc.subcore_barrier()
    pltpu.sync_copy(sh_sidx, sidxv)

    def perm_chunk(k, exts):
        extmx, extmy = exts
        sl = pl.ds(k * _L, _L)
        idx = sidxv[sl]
        gx0 = plsc.load_gather(x0v, [idx])
        gy0 = plsc.load_gather(y0v, [idx])
        gx1 = plsc.load_gather(x1v, [idx])
        gy1 = plsc.load_gather(y1v, [idx])
        gsc = plsc.load_gather(scv, [idx])
        gx0v[sl] = gx0
        gy0v[sl] = gy0
        gx1v[sl] = gx1
        gy1v[sl] = gy1
        gscv[sl] = gsc
        kbv[sl] = _bucketx(gx0) * _KY + _buckety(gy0)
        extx = gx1 - gx0
        exty = gy1 - gy0
        arv[sl] = extx * exty
        return (jnp.maximum(extmx, extx), jnp.maximum(extmy, exty))
    zf = jnp.zeros((_L,), jnp.float32)
    extmx, extmy = lax.fori_loop(0, _NC, perm_chunk, (zf, zf))
    extmx = _bfly(extmx, jnp.maximum)
    extmy = _bfly(extmy, jnp.maximum)

    def row_block(rb, carry):
        blk = wid + _NW * rb
        i0 = blk * _L
        sl_i = pl.ds(i0, _L)
        rx0 = gx0v[sl_i]
        ry0 = gy0v[sl_i]
        rx1 = gx1v[sl_i]
        ry1 = gy1v[sl_i]
        rar = arv[sl_i]
        rsc = gscv[sl_i]
        oidxv[pl.ds(rb * _L, _L)] = sidxv[sl_i]

        xlo = _bucketx(_bfly(rx0, jnp.minimum) - 0.6 * extmx - 0.5)[0]
        xhi = _bucketx(_bfly(0.6 * rx1 + 0.4 * rx0, jnp.maximum) + 0.5)[0]
        ylo = _buckety(_bfly(ry0, jnp.minimum) - 0.6 * extmy - 0.5)[0]
        yhi = _buckety(_bfly(0.6 * ry1 + 0.4 * ry0, jnp.maximum) + 0.5)[0]

        res = jnp.zeros((_L,), jnp.float32)
        for g in range(_L // _RU):
            ix0 = [rx0[g * _RU + r] for r in range(_RU)]
            iy0 = [ry0[g * _RU + r] for r in range(_RU)]
            ix1 = [rx1[g * _RU + r] for r in range(_RU)]
            iy1 = [ry1[g * _RU + r] for r in range(_RU)]
            iar = [rar[g * _RU + r] for r in range(_RU)]
            isc = [rsc[g * _RU + r] for r in range(_RU)]

            def strip(xb, accs):
                cl = xb * _KY + ylo
                ch = xb * _KY + yhi
                e_lo = gofs_s[cl]
                e_hi = gofs_s[ch + 1]
                c_lo = lax.shift_right_logical(e_lo, 4)
                c_hi = lax.shift_right_logical(e_hi + 15, 4)

                def jchunk(k, accs):
                    sl = pl.ds(k * _L, _L)
                    jx0 = gx0v[sl]
                    jy0 = gy0v[sl]
                    jx1 = gx1v[sl]
                    jy1 = gy1v[sl]
                    js = gscv[sl]
                    ja = arv[sl]
                    jkb = kbv[sl]
                    m = (jkb >= cl) & (jkb <= ch)
                    nxt = []
                    for r in range(_RU):
                        wx = jnp.maximum(jnp.minimum(ix1[r], jx1) - jnp.maximum(ix0[r], jx0), 0.0)
                        wy = jnp.maximum(jnp.minimum(iy1[r], jy1) - jnp.maximum(iy0[r], jy0), 0.0)
                        inter = wx * wy
                        union = iar[r] + ja - inter
                        hit = (inter > _THR * union) & (js > isc[r]) & m
                        ratio = (union - inter) / union
                        nxt.append(accs[r] * jnp.where(hit, ratio, 1.0))
                    return tuple(nxt)

                return lax.fori_loop(c_lo, c_hi, jchunk, accs)

            accs = lax.fori_loop(xlo, xhi + 1, strip, (ones,) * _RU)
            for r in range(_RU):
                a = _bfly(accs[r], jnp.multiply)
                res = jnp.where(lanes == (g * _RU + r), a[0], res)
        outv[pl.ds(rb * _L, _L)] = res * rsc
        return carry

    lax.fori_loop(0, _NB, row_block, 0)

    obufs = (ob0, ob1)
    for q in range(_RPW // _L):
        obufs[q // 5][pl.ds((q % 5) * _L, _L)] = oidxv[pl.ds(q * _L, _L)]
    for h in range(2):
        pltpu.async_copy(outv.at[pl.ds(h * 80, 80)], outh.at[obufs[h]], sem)
    pltpu.make_async_copy(outv.at[pl.ds(0, _RPW)],
                          outh.at[pl.ds(0, _RPW)], sem).wait()


_mesh = plsc.VectorSubcoreMesh(core_axis_name="c", subcore_axis_name="s")

_decay_call = pl.kernel(
    _decay_body,
    out_type=jax.ShapeDtypeStruct((_NP,), jnp.float32),
    mesh=_mesh,
    scratch_types=[
        pltpu.VMEM((_NP,), jnp.float32),
        pltpu.VMEM((_NP,), jnp.float32),
        pltpu.VMEM((_NP,), jnp.float32),
        pltpu.VMEM((_NP,), jnp.float32),
        pltpu.VMEM((_NP,), jnp.float32),
        pltpu.VMEM((_NP,), jnp.float32),
        pltpu.VMEM((_NP,), jnp.float32),
        pltpu.VMEM((_NP,), jnp.float32),
        pltpu.VMEM((_NP,), jnp.float32),
        pltpu.VMEM((_NP,), jnp.float32),
        pltpu.VMEM((_NP,), jnp.float32),
        pltpu.VMEM((_NP,), jnp.int32),
        pltpu.VMEM((_BPS,), jnp.int32),
        pltpu.VMEM((_BPS,), jnp.int32),
        pltpu.VMEM((_BPS,), jnp.int32),
        pltpu.VMEM((_NP,), jnp.int32),
        pltpu.VMEM((_RPW,), jnp.int32),
        pltpu.VMEM((_RPW,), jnp.float32),
        pltpu.VMEM((_K2,), jnp.int32),
        pltpu.VMEM((_K2,), jnp.int32),
        pltpu.VMEM((_NSC * _K2,), jnp.int32),
        pltpu.VMEM((80,), jnp.int32),
        pltpu.VMEM((80,), jnp.int32),
        pltpu.VMEM((80,), jnp.int32),
        pltpu.VMEM((80,), jnp.int32),
        pltpu.VMEM((80,), jnp.int32),
        pltpu.VMEM((80,), jnp.int32),
        pltpu.SMEM((_K2,), jnp.int32),
        pltpu.SMEM((_K2,), jnp.int32),
        pltpu.SMEM((_K2 + 1,), jnp.int32),
        pltpu.VMEM_SHARED((_NSC * _K2,), jnp.int32),
        pltpu.VMEM_SHARED((_NP,), jnp.int32),
        pltpu.SemaphoreType.DMA,
    ],
    compiler_params=pltpu.CompilerParams(needs_layout_passes=False),
)


def kernel(boxes, scores):
    pad = _NP - _N
    big = jnp.float32(4.0e8)
    x0 = jnp.concatenate([boxes[:, 0], jnp.full((pad,), big, jnp.float32)])
    y0 = jnp.concatenate([boxes[:, 1], jnp.full((pad,), big, jnp.float32)])
    x1 = jnp.concatenate([boxes[:, 2] + 1.0, jnp.full((pad,), big + 1.0, jnp.float32)])
    y1 = jnp.concatenate([boxes[:, 3] + 1.0, jnp.full((pad,), big + 1.0, jnp.float32)])
    sc = jnp.concatenate([scores, jnp.zeros((pad,), jnp.float32)])
    decayed = _decay_call(x0, y0, x1, y1, sc)
    return decayed[:_N]
